# baseline (speedup 1.0000x reference)
import functools

import jax
import jax.numpy as jnp
from jax import lax
from jax.experimental import pallas as pl
from jax.experimental.pallas import tpu as pltpu
from jax.experimental.pallas import tpu_sc as plsc

_R, _C = 1024, 4096
_N = _R * _C
_GRID = 8
_BR = _R // _GRID

_NC, _NS, _L = 2, 16, 16
_NW = _NC * _NS
_PER_W = _N // _NW
_ROWS_W = _R // _NW
_CROWS = 2
_NCH = _ROWS_W // _CROWS


def _bce_body(lp_ref, lt_ref, dp_ref, dt_ref, bits_ref, scal_ref, acc_ref):
    i = pl.program_id(0)

    dt = dt_ref[...]
    dp = dp_ref[...]
    pos = dt == 1.0
    neg = dt == 0.0
    q = jnp.where(pos, dp, 1.0 - dp)
    bce = -jnp.clip(jnp.log(q), -100.0, None)

    bits_ref[...] = jnp.where(
        neg, lax.bitcast_convert_type(bce, jnp.int32), jnp.int32(0))

    @pl.when(i == 0)
    def _init():
        acc_ref[0] = 0.0
        acc_ref[1] = 0.0
        acc_ref[2] = 0.0

    acc_ref[0] += jnp.sum(jnp.where(pos, bce, 0.0))
    acc_ref[1] += jnp.sum(pos.astype(jnp.int32)).astype(jnp.float32)
    acc_ref[2] += jnp.sum(neg.astype(jnp.int32)).astype(jnp.float32)

    @pl.when(i == _GRID - 1)
    def _fin():
        d = lp_ref[...] - lt_ref[...]
        n_neg = acc_ref[2]
        scal_ref[0] = acc_ref[0]
        scal_ref[1] = acc_ref[1]
        scal_ref[2] = n_neg
        scal_ref[3] = jnp.mean(d * d)
        scal_ref[4] = jnp.floor(0.7 * n_neg)
        scal_ref[5] = 0.0
        scal_ref[6] = 0.0
        scal_ref[7] = 0.0


def _stage_a(label_p, label_t, dp, dt):
    return pl.pallas_call(
        _bce_body,
        grid=(_GRID,),
        in_specs=[
            pl.BlockSpec((_R, 4), lambda i: (0, 0)),
            pl.BlockSpec((_R, 4), lambda i: (0, 0)),
            pl.BlockSpec((_BR, _C), lambda i: (i, 0)),
            pl.BlockSpec((_BR, _C), lambda i: (i, 0)),
        ],
        out_shape=[
            jax.ShapeDtypeStruct((_R, _C), jnp.int32),
            jax.ShapeDtypeStruct((8,), jnp.float32),
        ],
        out_specs=[
            pl.BlockSpec((_BR, _C), lambda i: (i, 0)),
            pl.BlockSpec(memory_space=pltpu.SMEM),
        ],
        scratch_shapes=[pltpu.SMEM((4,), jnp.float32)],
    )(label_p, label_t, dp, dt)


_MESH = plsc.VectorSubcoreMesh(core_axis_name="c", subcore_axis_name="s")


@functools.partial(
    pl.kernel,
    mesh=_MESH,
    compiler_params=pltpu.CompilerParams(needs_layout_passes=False),
    out_type=jax.ShapeDtypeStruct((_NW, 2, 32768), jnp.int32),
    scratch_types=[
        pltpu.VMEM((_CROWS, _C), jnp.int32),
        pltpu.VMEM((_CROWS, _C), jnp.int32),
        pltpu.VMEM((2, 32768), jnp.int32),
        pltpu.SemaphoreType.DMA,
        pltpu.SemaphoreType.DMA,
    ],
)
def _hist_hi(bits_hbm, hist_hbm, buf0, buf1, hist_v, sem0, sem1):
    wid = lax.axis_index("s") * _NC + lax.axis_index("c")
    base = wid * _ROWS_W
    bufs, sems = (buf0, buf1), (sem0, sem1)

    for p in range(2):
        @plsc.parallel_loop(0, 32768 // _L, unroll=16)
        def _zero(i):
            hist_v[p, pl.ds(i * _L, _L)] = jnp.zeros((_L,), jnp.int32)

    ones = jnp.ones((_L,), jnp.int32)
    parity = lax.iota(jnp.int32, _L) & 1

    def cp(c, b):
        return pltpu.make_async_copy(
            bits_hbm.at[pl.ds(base + c * _CROWS, _CROWS)], bufs[b], sems[b])

    cp(0, 0).start()
    cp(1, 1).start()

    def outer(g0, _):
        for b in range(2):
            g = g0 * 2 + b
            cp(g, b).wait()

            for rr in range(_CROWS):
                @plsc.parallel_loop(0, _C // _L, unroll=16)
                def _inner(j):
                    vec = bufs[b][rr, pl.ds(j * _L, _L)]
                    binv = lax.shift_right_logical(vec, 16)
                    plsc.addupdate_scatter(hist_v, [parity, binv], ones,
                                           mask=binv > 0)

            @pl.when(g + 2 < _NCH)
            def _next():
                cp(g + 2, b).start()
        return 0
    lax.fori_loop(0, _NCH // 2, outer, 0)

    pltpu.sync_copy(hist_v, hist_hbm.at[wid])


def _suffix_counts(hf):
    rows = hf.shape[0]
    ci = lax.broadcasted_iota(jnp.int32, (128, 128), 0)
    cj = lax.broadcasted_iota(jnp.int32, (128, 128), 1)
    u = (ci > cj).astype(jnp.float32)
    ws = jax.lax.dot_general(hf, u, (((1,), (0,)), ((), ())),
                             preferred_element_type=jnp.float32)
    ri = lax.broadcasted_iota(jnp.int32, (rows, rows), 0)
    rj = lax.broadcasted_iota(jnp.int32, (rows, rows), 1)
    a = (rj > ri).astype(jnp.float32)
    rowsum = jnp.sum(hf, axis=1, keepdims=True)
    rowsuf = jax.lax.dot_general(a, rowsum, (((1,), (0,)), ((), ())),
                                 preferred_element_type=jnp.float32)
    return ws + rowsuf


def _pick_body(hist_ref, scal_ref, out_ref):
    h = jnp.sum(hist_ref[...].reshape(64, 256, 128), axis=0)
    rc = _suffix_counts(h.astype(jnp.float32))
    k = scal_ref[4]
    bstar = jnp.sum((rc >= k).astype(jnp.int32))
    ri = lax.broadcasted_iota(jnp.int32, (256, 128), 0)
    cj = lax.broadcasted_iota(jnp.int32, (256, 128), 1)
    flat = ri * 128 + cj
    count_above = jnp.sum(jnp.where(flat == bstar, rc, 0.0))
    out_ref[0] = bstar.astype(jnp.float32)
    out_ref[1] = count_above


def _stage_c(hist, scal):
    return pl.pallas_call(
        _pick_body,
        in_specs=[
            pl.BlockSpec((16384, 128), lambda: (0, 0)),
            pl.BlockSpec(memory_space=pltpu.SMEM),
        ],
        out_shape=jax.ShapeDtypeStruct((8,), jnp.float32),
        out_specs=pl.BlockSpec(memory_space=pltpu.SMEM),
    )(hist, scal)


@functools.partial(
    pl.kernel,
    mesh=_MESH,
    compiler_params=pltpu.CompilerParams(needs_layout_passes=False),
    out_type=[
        jax.ShapeDtypeStruct((_NW, 65536), jnp.int32),
        jax.ShapeDtypeStruct((_NW, _L), jnp.float32),
    ],
    scratch_types=[
        pltpu.VMEM((_CROWS, _C), jnp.int32),
        pltpu.VMEM((_CROWS, _C), jnp.int32),
        pltpu.VMEM((65536,), jnp.int32),
        pltpu.VMEM((_L,), jnp.int32),
        pltpu.VMEM((_L,), jnp.float32),
        pltpu.SemaphoreType.DMA,
        pltpu.SemaphoreType.DMA,
    ],
)
def _hist_lo(bits_hbm, bstar_hbm, hist_hbm, sums_hbm,
             buf0, buf1, hist_v, bv, sv, sem0, sem1):
    wid = lax.axis_index("s") * _NC + lax.axis_index("c")
    base = wid * _ROWS_W
    bufs, sems = (buf0, buf1), (sem0, sem1)

    pltpu.sync_copy(bstar_hbm, bv)
    bstar = bv[...]

    @plsc.parallel_loop(0, 65536 // _L, unroll=16)
    def _zero(i):
        hist_v[pl.ds(i * _L, _L)] = jnp.zeros((_L,), jnp.int32)

    ones = jnp.ones((_L,), jnp.int32)
    lo_mask = jnp.int32(0xFFFF)

    def cp(c, b):
        return pltpu.make_async_copy(
            bits_hbm.at[pl.ds(base + c * _CROWS, _CROWS)], bufs[b], sems[b])

    cp(0, 0).start()
    cp(1, 1).start()

    def outer(g0, acc):
        for b in range(2):
            g = g0 * 2 + b
            cp(g, b).wait()

            for rr in range(_CROWS):
                @plsc.parallel_loop(0, _C // _L, unroll=16, carry=acc)
                def _inner(j, a):
                    vec = bufs[b][rr, pl.ds(j * _L, _L)]
                    hi = lax.shift_right_logical(vec, 16)
                    match = hi == bstar
                    low = vec & lo_mask
                    plsc.addupdate_scatter(hist_v, [low], ones, mask=match)
                    vals = plsc.bitcast(vec, jnp.float32)
                    return a + jnp.where(hi > bstar, vals, 0.0)
                acc = _inner

            @pl.when(g + 2 < _NCH)
            def _next():
                cp(g + 2, b).start()
        return acc

    acc = lax.fori_loop(0, _NCH // 2, outer, jnp.zeros((_L,), jnp.float32))
    sv[...] = acc

    pltpu.sync_copy(hist_v, hist_hbm.at[wid])
    pltpu.sync_copy(sv, sums_hbm.at[wid])


def _final_body(hist_ref, sums_ref, scal_a_ref, scal_c_ref, out_ref):
    h2 = jnp.sum(hist_ref[...].reshape(32, 512, 128), axis=0)
    rc2 = _suffix_counts(h2.astype(jnp.float32))
    k = scal_a_ref[4]
    count_above = scal_c_ref[1]
    bstar_i = scal_c_ref[0].astype(jnp.int32)

    lstar = jnp.sum((count_above + rc2 >= k).astype(jnp.int32))
    ri = lax.broadcasted_iota(jnp.int32, (512, 128), 0)
    cj = lax.broadcasted_iota(jnp.int32, (512, 128), 1)
    flat = ri * 128 + cj
    vbits = bstar_i * 65536 + flat
    vvals = lax.bitcast_convert_type(vbits, jnp.float32)
    h2f = h2.astype(jnp.float32)
    above = flat > lstar
    inbin_cnt = jnp.sum(jnp.where(above, h2f, 0.0))
    inbin_sum = jnp.sum(jnp.where(above, h2f * vvals, 0.0))

    sum_hi = jnp.sum(sums_ref[...])
    vstar = lax.bitcast_convert_type(bstar_i * 65536 + lstar, jnp.float32)
    cnt_tot = count_above + inbin_cnt
    loss_neg = (sum_hi + inbin_sum + (k - cnt_tot) * vstar) / k
    loss_pos = scal_a_ref[0] / scal_a_ref[1]
    out_ref[0, 0] = scal_a_ref[3] + loss_pos + loss_neg


def _stage_e(hist2, sums, scal_a, scal_c):
    return pl.pallas_call(
        _final_body,
        in_specs=[
            pl.BlockSpec((16384, 128), lambda: (0, 0)),
            pl.BlockSpec((_NW, _L), lambda: (0, 0)),
            pl.BlockSpec(memory_space=pltpu.SMEM),
            pl.BlockSpec(memory_space=pltpu.SMEM),
        ],
        out_shape=jax.ShapeDtypeStruct((1, 1), jnp.float32),
        out_specs=pl.BlockSpec(memory_space=pltpu.SMEM),
    )(hist2, sums, scal_a, scal_c)


def kernel(label_p, label_t, denselabel_p, denselabel_t):
    bits, scal_a = _stage_a(label_p, label_t, denselabel_p, denselabel_t)
    hist1 = _hist_hi(bits)
    scal_c = _stage_c(hist1.reshape(16384, 128), scal_a)
    bvec = jnp.broadcast_to(scal_c[0].astype(jnp.int32), (_L,))
    hist2, sums = _hist_lo(bits, bvec)
    out = _stage_e(hist2.reshape(16384, 128), sums, scal_a, scal_c)
    return out[0, 0]


# speedup vs baseline: 1.1399x; 1.1399x over previous
import functools

import jax
import jax.numpy as jnp
from jax import lax
from jax.experimental import pallas as pl
from jax.experimental.pallas import tpu as pltpu
from jax.experimental.pallas import tpu_sc as plsc

_R, _C = 1024, 4096
_N = _R * _C
_GRID = 8
_BR = _R // _GRID

_NC, _NS, _L = 2, 16, 16
_NW = _NC * _NS
_PER_W = _N // _NW
_ROWS_W = _R // _NW
_CROWS = 4
_NCH = _ROWS_W // _CROWS


def _bce_body(lp_ref, lt_ref, dp_ref, dt_ref, bits_ref, scal_ref, acc_ref):
    i = pl.program_id(0)

    dt = dt_ref[...]
    dp = dp_ref[...]
    pos = dt == 1.0
    neg = dt == 0.0
    q = jnp.where(pos, dp, 1.0 - dp)
    bce = -jnp.clip(jnp.log(q), -100.0, None)

    bits_ref[...] = jnp.where(
        neg, lax.bitcast_convert_type(bce, jnp.int32), jnp.int32(0))

    @pl.when(i == 0)
    def _init():
        acc_ref[0] = 0.0
        acc_ref[1] = 0.0
        acc_ref[2] = 0.0

    acc_ref[0] += jnp.sum(jnp.where(pos, bce, 0.0))
    acc_ref[1] += jnp.sum(pos.astype(jnp.int32)).astype(jnp.float32)
    acc_ref[2] += jnp.sum(neg.astype(jnp.int32)).astype(jnp.float32)

    @pl.when(i == _GRID - 1)
    def _fin():
        d = lp_ref[...] - lt_ref[...]
        n_neg = acc_ref[2]
        scal_ref[0] = acc_ref[0]
        scal_ref[1] = acc_ref[1]
        scal_ref[2] = n_neg
        scal_ref[3] = jnp.mean(d * d)
        scal_ref[4] = jnp.floor(0.7 * n_neg)
        scal_ref[5] = 0.0
        scal_ref[6] = 0.0
        scal_ref[7] = 0.0


def _stage_a(label_p, label_t, dp, dt):
    return pl.pallas_call(
        _bce_body,
        grid=(_GRID,),
        in_specs=[
            pl.BlockSpec((_R, 4), lambda i: (0, 0)),
            pl.BlockSpec((_R, 4), lambda i: (0, 0)),
            pl.BlockSpec((_BR, _C), lambda i: (i, 0)),
            pl.BlockSpec((_BR, _C), lambda i: (i, 0)),
        ],
        out_shape=[
            jax.ShapeDtypeStruct((_R, _C), jnp.int32),
            jax.ShapeDtypeStruct((8,), jnp.float32),
        ],
        out_specs=[
            pl.BlockSpec((_BR, _C), lambda i: (i, 0)),
            pl.BlockSpec(memory_space=pltpu.SMEM),
        ],
        scratch_shapes=[pltpu.SMEM((4,), jnp.float32)],
    )(label_p, label_t, dp, dt)


_MESH = plsc.VectorSubcoreMesh(core_axis_name="c", subcore_axis_name="s")


@functools.partial(
    pl.kernel,
    mesh=_MESH,
    compiler_params=pltpu.CompilerParams(needs_layout_passes=False),
    out_type=jax.ShapeDtypeStruct((_NW, 32768), jnp.int32),
    scratch_types=[
        pltpu.VMEM((_CROWS, _C), jnp.int32),
        pltpu.VMEM((_CROWS, _C), jnp.int32),
        pltpu.VMEM((32768,), jnp.int32),
        pltpu.SemaphoreType.DMA,
        pltpu.SemaphoreType.DMA,
    ],
)
def _hist_hi(bits_hbm, hist_hbm, buf0, buf1, hist_v, sem0, sem1):
    wid = lax.axis_index("s") * _NC + lax.axis_index("c")
    base = wid * _ROWS_W
    bufs, sems = (buf0, buf1), (sem0, sem1)

    @plsc.parallel_loop(0, 32768 // _L, unroll=16)
    def _zero(i):
        hist_v[pl.ds(i * _L, _L)] = jnp.zeros((_L,), jnp.int32)

    ones = jnp.ones((_L,), jnp.int32)

    def cp(c, b):
        return pltpu.make_async_copy(
            bits_hbm.at[pl.ds(base + c * _CROWS, _CROWS)], bufs[b], sems[b])

    cp(0, 0).start()
    cp(1, 1).start()

    def outer(g0, _):
        for b in range(2):
            g = g0 * 2 + b
            cp(g, b).wait()

            for rr in range(_CROWS):
                @plsc.parallel_loop(0, _C // _L, unroll=16)
                def _inner(j):
                    vec = bufs[b][rr, pl.ds(j * _L, _L)]
                    binv = lax.shift_right_logical(vec, 16)
                    plsc.addupdate_scatter(hist_v, [binv], ones,
                                           mask=binv > 0)

            @pl.when(g + 2 < _NCH)
            def _next():
                cp(g + 2, b).start()
        return 0
    lax.fori_loop(0, _NCH // 2, outer, 0)

    pltpu.sync_copy(hist_v, hist_hbm.at[wid])


def _suffix_counts(hf):
    rows = hf.shape[0]
    ci = lax.broadcasted_iota(jnp.int32, (128, 128), 0)
    cj = lax.broadcasted_iota(jnp.int32, (128, 128), 1)
    u = (ci > cj).astype(jnp.float32)
    ws = jax.lax.dot_general(hf, u, (((1,), (0,)), ((), ())),
                             preferred_element_type=jnp.float32)
    ri = lax.broadcasted_iota(jnp.int32, (rows, rows), 0)
    rj = lax.broadcasted_iota(jnp.int32, (rows, rows), 1)
    a = (rj > ri).astype(jnp.float32)
    rowsum = jnp.sum(hf, axis=1, keepdims=True)
    rowsuf = jax.lax.dot_general(a, rowsum, (((1,), (0,)), ((), ())),
                                 preferred_element_type=jnp.float32)
    return ws + rowsuf


def _pick_body(hist_ref, scal_ref, out_ref):
    h = jnp.sum(hist_ref[...].reshape(32, 256, 128), axis=0)
    rc = _suffix_counts(h.astype(jnp.float32))
    k = scal_ref[4]
    bstar = jnp.sum((rc >= k).astype(jnp.int32))
    ri = lax.broadcasted_iota(jnp.int32, (256, 128), 0)
    cj = lax.broadcasted_iota(jnp.int32, (256, 128), 1)
    flat = ri * 128 + cj
    count_above = jnp.sum(jnp.where(flat == bstar, rc, 0.0))
    out_ref[0] = bstar.astype(jnp.float32)
    out_ref[1] = count_above


def _stage_c(hist, scal):
    return pl.pallas_call(
        _pick_body,
        in_specs=[
            pl.BlockSpec((8192, 128), lambda: (0, 0)),
            pl.BlockSpec(memory_space=pltpu.SMEM),
        ],
        out_shape=jax.ShapeDtypeStruct((8,), jnp.float32),
        out_specs=pl.BlockSpec(memory_space=pltpu.SMEM),
    )(hist, scal)


@functools.partial(
    pl.kernel,
    mesh=_MESH,
    compiler_params=pltpu.CompilerParams(needs_layout_passes=False),
    out_type=[
        jax.ShapeDtypeStruct((_NW, 65536), jnp.int32),
        jax.ShapeDtypeStruct((_NW, _L), jnp.float32),
    ],
    scratch_types=[
        pltpu.VMEM((_CROWS, _C), jnp.int32),
        pltpu.VMEM((_CROWS, _C), jnp.int32),
        pltpu.VMEM((65536,), jnp.int32),
        pltpu.VMEM((_L,), jnp.int32),
        pltpu.VMEM((_L,), jnp.float32),
        pltpu.SemaphoreType.DMA,
        pltpu.SemaphoreType.DMA,
    ],
)
def _hist_lo(bits_hbm, bstar_hbm, hist_hbm, sums_hbm,
             buf0, buf1, hist_v, bv, sv, sem0, sem1):
    wid = lax.axis_index("s") * _NC + lax.axis_index("c")
    base = wid * _ROWS_W
    bufs, sems = (buf0, buf1), (sem0, sem1)

    pltpu.sync_copy(bstar_hbm, bv)
    bstar = bv[...]

    @plsc.parallel_loop(0, 65536 // _L, unroll=16)
    def _zero(i):
        hist_v[pl.ds(i * _L, _L)] = jnp.zeros((_L,), jnp.int32)

    ones = jnp.ones((_L,), jnp.int32)
    lo_mask = jnp.int32(0xFFFF)

    def cp(c, b):
        return pltpu.make_async_copy(
            bits_hbm.at[pl.ds(base + c * _CROWS, _CROWS)], bufs[b], sems[b])

    cp(0, 0).start()
    cp(1, 1).start()

    def outer(g0, acc):
        for b in range(2):
            g = g0 * 2 + b
            cp(g, b).wait()

            for rr in range(_CROWS):
                @plsc.parallel_loop(0, _C // _L, unroll=16, carry=acc)
                def _inner(j, a):
                    vec = bufs[b][rr, pl.ds(j * _L, _L)]
                    hi = lax.shift_right_logical(vec, 16)
                    match = hi == bstar
                    low = vec & lo_mask
                    plsc.addupdate_scatter(hist_v, [low], ones, mask=match)
                    vals = plsc.bitcast(vec, jnp.float32)
                    return a + jnp.where(hi > bstar, vals, 0.0)
                acc = _inner

            @pl.when(g + 2 < _NCH)
            def _next():
                cp(g + 2, b).start()
        return acc

    acc = lax.fori_loop(0, _NCH // 2, outer, jnp.zeros((_L,), jnp.float32))
    sv[...] = acc

    pltpu.sync_copy(hist_v, hist_hbm.at[wid])
    pltpu.sync_copy(sv, sums_hbm.at[wid])


def _final_body(hist_ref, sums_ref, scal_a_ref, scal_c_ref, out_ref):
    h2 = jnp.sum(hist_ref[...].reshape(32, 512, 128), axis=0)
    rc2 = _suffix_counts(h2.astype(jnp.float32))
    k = scal_a_ref[4]
    count_above = scal_c_ref[1]
    bstar_i = scal_c_ref[0].astype(jnp.int32)

    lstar = jnp.sum((count_above + rc2 >= k).astype(jnp.int32))
    ri = lax.broadcasted_iota(jnp.int32, (512, 128), 0)
    cj = lax.broadcasted_iota(jnp.int32, (512, 128), 1)
    flat = ri * 128 + cj
    vbits = bstar_i * 65536 + flat
    vvals = lax.bitcast_convert_type(vbits, jnp.float32)
    h2f = h2.astype(jnp.float32)
    above = flat > lstar
    inbin_cnt = jnp.sum(jnp.where(above, h2f, 0.0))
    inbin_sum = jnp.sum(jnp.where(above, h2f * vvals, 0.0))

    sum_hi = jnp.sum(sums_ref[...])
    vstar = lax.bitcast_convert_type(bstar_i * 65536 + lstar, jnp.float32)
    cnt_tot = count_above + inbin_cnt
    loss_neg = (sum_hi + inbin_sum + (k - cnt_tot) * vstar) / k
    loss_pos = scal_a_ref[0] / scal_a_ref[1]
    out_ref[0, 0] = scal_a_ref[3] + loss_pos + loss_neg


def _stage_e(hist2, sums, scal_a, scal_c):
    return pl.pallas_call(
        _final_body,
        in_specs=[
            pl.BlockSpec((16384, 128), lambda: (0, 0)),
            pl.BlockSpec((_NW, _L), lambda: (0, 0)),
            pl.BlockSpec(memory_space=pltpu.SMEM),
            pl.BlockSpec(memory_space=pltpu.SMEM),
        ],
        out_shape=jax.ShapeDtypeStruct((1, 1), jnp.float32),
        out_specs=pl.BlockSpec(memory_space=pltpu.SMEM),
    )(hist2, sums, scal_a, scal_c)


def kernel(label_p, label_t, denselabel_p, denselabel_t):
    bits, scal_a = _stage_a(label_p, label_t, denselabel_p, denselabel_t)
    hist1 = _hist_hi(bits)
    scal_c = _stage_c(hist1.reshape(8192, 128), scal_a)
    bvec = jnp.broadcast_to(scal_c[0].astype(jnp.int32), (_L,))
    hist2, sums = _hist_lo(bits, bvec)
    out = _stage_e(hist2.reshape(16384, 128), sums, scal_a, scal_c)
    return out[0, 0]
